# baseline (speedup 1.0000x reference)
import functools

import jax
import jax.numpy as jnp
from jax import lax
from jax.experimental import pallas as pl
from jax.experimental.pallas import tpu as pltpu
from jax.experimental.pallas import tpu_sc as plsc


_CH = 8


def _sc_add_kernel(batch, seq_len, dim, x_hbm, w_hbm, out_hbm, xbuf, wbuf, sem):
    nc = 2
    ns = 16
    wid = lax.axis_index("s") * nc + lax.axis_index("c")
    rows_per_worker = seq_len // (nc * ns)
    s_base = wid * rows_per_worker
    n_chunks = rows_per_worker // _CH
    vecs_per_row = dim // 16
    for b in range(batch):
        def body(j, carry):
            s0 = s_base + j * _CH
            pltpu.sync_copy(x_hbm.at[b, pl.ds(s0, _CH)], xbuf)
            pltpu.sync_copy(w_hbm.at[pl.ds(s0, _CH)], wbuf)

            def add_row(r, c2):
                def add_vec(v, c3):
                    off = v * 16
                    plsc.addupdate(
                        xbuf.at[r, pl.ds(off, 16)], wbuf[r, pl.ds(off, 16)]
                    )
                    return c3
                lax.fori_loop(0, vecs_per_row, add_vec, 0)
                return c2
            lax.fori_loop(0, _CH, add_row, 0)

            pltpu.sync_copy(xbuf, out_hbm.at[b, pl.ds(s0, _CH)])
            return carry
        lax.fori_loop(0, n_chunks, body, 0)


def kernel(inputs, weight):
    batch, seq_len, dim = inputs.shape
    mesh = plsc.VectorSubcoreMesh(core_axis_name="c", subcore_axis_name="s")
    k = pl.kernel(
        functools.partial(_sc_add_kernel, batch, seq_len, dim),
        out_type=jax.ShapeDtypeStruct((batch, seq_len, dim), inputs.dtype),
        mesh=mesh,
        scratch_types=[
            pltpu.VMEM((_CH, dim), jnp.float32),
            pltpu.VMEM((_CH, dim), jnp.float32),
            pltpu.SemaphoreType.DMA,
        ],
    )
    return k(inputs, weight)


# speedup vs baseline: 1.6754x; 1.6754x over previous
import functools

import jax
import jax.numpy as jnp
from jax import lax
from jax.experimental import pallas as pl
from jax.experimental.pallas import tpu as pltpu
from jax.experimental.pallas import tpu_sc as plsc


_CH = 8


def _sc_add_kernel(batch, seq_len, dim, x_hbm, w_hbm, out_hbm, xbuf, wbuf, sem):
    nc = 2
    ns = 16
    wid = lax.axis_index("s") * nc + lax.axis_index("c")
    rows_per_worker = seq_len // (nc * ns)
    s_base = wid * rows_per_worker
    n_chunks = rows_per_worker // _CH
    vecs_per_row = dim // 16
    for b in range(batch):
        def body(j, carry):
            s0 = s_base + j * _CH
            pltpu.sync_copy(x_hbm.at[b, pl.ds(s0, _CH)], xbuf)
            pltpu.sync_copy(w_hbm.at[pl.ds(s0, _CH)], wbuf)

            def add_row(r, c2):
                def add_vec(v, c3):
                    off = v * 16
                    plsc.addupdate(
                        xbuf.at[r, pl.ds(off, 16)], wbuf[r, pl.ds(off, 16)]
                    )
                    return c3
                lax.fori_loop(0, vecs_per_row, add_vec, 0, unroll=8)
                return c2
            lax.fori_loop(0, _CH, add_row, 0)

            pltpu.sync_copy(xbuf, out_hbm.at[b, pl.ds(s0, _CH)])
            return carry
        lax.fori_loop(0, n_chunks, body, 0)


def kernel(inputs, weight):
    batch, seq_len, dim = inputs.shape
    mesh = plsc.VectorSubcoreMesh(core_axis_name="c", subcore_axis_name="s")
    k = pl.kernel(
        functools.partial(_sc_add_kernel, batch, seq_len, dim),
        out_type=jax.ShapeDtypeStruct((batch, seq_len, dim), inputs.dtype),
        mesh=mesh,
        scratch_types=[
            pltpu.VMEM((_CH, dim), jnp.float32),
            pltpu.VMEM((_CH, dim), jnp.float32),
            pltpu.SemaphoreType.DMA,
        ],
    )
    return k(inputs, weight)


# speedup vs baseline: 1.8882x; 1.1270x over previous
import functools

import jax
import jax.numpy as jnp
from jax import lax
from jax.experimental import pallas as pl
from jax.experimental.pallas import tpu as pltpu
from jax.experimental.pallas import tpu_sc as plsc


_CH = 8


def _sc_add_kernel(batch, seq_len, dim, x_hbm, w_hbm, out_hbm, xbuf, wbuf, sem):
    nc = 2
    ns = 16
    wid = lax.axis_index("s") * nc + lax.axis_index("c")
    rows_per_worker = seq_len // (nc * ns)
    s_base = wid * rows_per_worker
    n_chunks = rows_per_worker // _CH
    vecs_per_row = dim // 16

    def body(j, carry):
        s0 = s_base + j * _CH
        pltpu.sync_copy(w_hbm.at[pl.ds(s0, _CH)], wbuf)
        for b in range(batch):
            pltpu.sync_copy(x_hbm.at[b, pl.ds(s0, _CH)], xbuf)

            def add_row(r, c2):
                def add_vec(v, c3):
                    off = v * 16
                    plsc.addupdate(
                        xbuf.at[r, pl.ds(off, 16)], wbuf[r, pl.ds(off, 16)]
                    )
                    return c3
                lax.fori_loop(0, vecs_per_row, add_vec, 0, unroll=16)
                return c2
            lax.fori_loop(0, _CH, add_row, 0)

            pltpu.sync_copy(xbuf, out_hbm.at[b, pl.ds(s0, _CH)])
        return carry
    lax.fori_loop(0, n_chunks, body, 0)


def kernel(inputs, weight):
    batch, seq_len, dim = inputs.shape
    mesh = plsc.VectorSubcoreMesh(core_axis_name="c", subcore_axis_name="s")
    k = pl.kernel(
        functools.partial(_sc_add_kernel, batch, seq_len, dim),
        out_type=jax.ShapeDtypeStruct((batch, seq_len, dim), inputs.dtype),
        mesh=mesh,
        scratch_types=[
            pltpu.VMEM((_CH, dim), jnp.float32),
            pltpu.VMEM((_CH, dim), jnp.float32),
            pltpu.SemaphoreType.DMA,
        ],
    )
    return k(inputs, weight)


# speedup vs baseline: 4.3533x; 2.3056x over previous
import jax
import jax.numpy as jnp
from jax.experimental import pallas as pl


_BLOCK_S = 256


def _add_kernel(x_ref, w_ref, o_ref):
    o_ref[...] = x_ref[...] + w_ref[...][None, :, :]


def kernel(inputs, weight):
    batch, seq_len, dim = inputs.shape
    bs = min(_BLOCK_S, seq_len)
    grid = (seq_len // bs,)
    return pl.pallas_call(
        _add_kernel,
        grid=grid,
        in_specs=[
            pl.BlockSpec((batch, bs, dim), lambda i: (0, i, 0)),
            pl.BlockSpec((bs, dim), lambda i: (i, 0)),
        ],
        out_specs=pl.BlockSpec((batch, bs, dim), lambda i: (0, i, 0)),
        out_shape=jax.ShapeDtypeStruct((batch, seq_len, dim), inputs.dtype),
    )(inputs, weight)
